# baseline (speedup 1.0000x reference)
import functools

import jax
import jax.numpy as jnp
from jax import lax
from jax.experimental import pallas as pl
from jax.experimental.pallas import tpu as pltpu
from jax.experimental.pallas import tpu_sc as plsc

T = 32768
B = 16
VEC = 64
NC = 2
NS = 16
NW = NC * NS
TW = T // NW
PHASES = 4
PT = TW // PHASES
GROUP = 128
RG = 2 * PT // GROUP
WG = PT // GROUP


def _sc_partials(feat2d, intr2d, divs, segs, vecs, intr_w, intr_b):
    mesh = plsc.VectorSubcoreMesh(core_axis_name="c", subcore_axis_name="s")

    @functools.partial(
        pl.kernel,
        out_type=jax.ShapeDtypeStruct((NW, B, 16), jnp.float32),
        mesh=mesh,
        compiler_params=pltpu.CompilerParams(use_tc_tiling_on_sc=False),
        scratch_types=[
            pltpu.VMEM((2 * TW // GROUP, GROUP), jnp.int32),
            pltpu.VMEM((TW // GROUP, GROUP), jnp.int32),
            pltpu.VMEM((TW + 16,), jnp.float32),
            pltpu.VMEM((TW + 16,), jnp.int32),
            pltpu.VMEM((16,), jnp.float32),
            pltpu.VMEM((2, 2 * PT, VEC), jnp.float32),
            pltpu.VMEM((2, PT + 16), jnp.float32),
            pltpu.VMEM((4, B, 16), jnp.float32),
            pltpu.VMEM((B, 16), jnp.float32),
            pltpu.SemaphoreType.DMA,
            pltpu.SemaphoreType.DMA,
        ],
    )
    def body(feat_hbm, intr_hbm, divs_hbm, segs_hbm, vecs_hbm, w_hbm, b_hbm,
             out_hbm, fidx_v, iidx_v, divs_v, segs_v, b_v, rows_v, w_v,
             acc_v, out_v, sem0, sem1):
        wid = lax.axis_index("c") * NS + lax.axis_index("s")
        sems = (sem0, sem1)

        pltpu.sync_copy(feat_hbm.at[pl.ds(wid * (2 * TW // GROUP),
                                          2 * TW // GROUP)], fidx_v)
        pltpu.sync_copy(intr_hbm.at[pl.ds(wid * (TW // GROUP),
                                          TW // GROUP)], iidx_v)
        pltpu.sync_copy(divs_hbm.at[pl.ds(wid * TW, TW)], divs_v.at[pl.ds(0, TW)])
        pltpu.sync_copy(segs_hbm.at[pl.ds(wid * TW, TW)], segs_v.at[pl.ds(0, TW)])
        pltpu.sync_copy(b_hbm, b_v)

        zero = jnp.zeros((16,), jnp.float32)
        for i in range(4):
            for s in range(B):
                acc_v[i, s, :] = zero

        bvec = b_v[...] * (1.0 / 16.0)

        def fire(p):
            buf = p % 2
            hs = []
            for j in range(RG):
                hs.append(pltpu.async_copy(
                    vecs_hbm.at[fidx_v.at[RG * p + j]],
                    rows_v.at[buf, pl.ds(j * GROUP, GROUP)],
                    sems[buf]))
            for j in range(WG):
                hs.append(pltpu.async_copy(
                    w_hbm.at[iidx_v.at[WG * p + j]],
                    w_v.at[buf, pl.ds(j * GROUP, GROUP)],
                    sems[buf]))
            return hs

        def compute(p):
            buf = p % 2

            def grp(gi, _):
                base = pl.multiple_of(gi * 16, 16)
                gbase = pl.multiple_of(p * PT + base, 16)
                cv = w_v[buf, pl.ds(base, 16)] / divs_v[pl.ds(gbase, 16)]
                sv = segs_v[pl.ds(gbase, 16)]
                for k in range(16):
                    i2 = 2 * (base + k)
                    s = (rows_v[buf, i2, pl.ds(0, 16)] * rows_v[buf, i2 + 1, pl.ds(0, 16)]
                         + rows_v[buf, i2, pl.ds(16, 16)] * rows_v[buf, i2 + 1, pl.ds(16, 16)])
                    s = s + (rows_v[buf, i2, pl.ds(32, 16)] * rows_v[buf, i2 + 1, pl.ds(32, 16)]
                             + rows_v[buf, i2, pl.ds(48, 16)] * rows_v[buf, i2 + 1, pl.ds(48, 16)])
                    sg = sv[k]
                    acc_v[k & 3, sg, :] = (acc_v[k & 3, sg, :]
                                           + (s * jnp.full((16,), cv[k], jnp.float32) + bvec))
                return 0

            lax.fori_loop(0, PT // 16, grp, 0)

        pending = fire(0)
        for p in range(PHASES):
            nxt = fire(p + 1) if p + 1 < PHASES else []
            for h in pending:
                h.wait()
            compute(p)
            pending = nxt

        for s in range(B):
            out_v[s, :] = ((acc_v[0, s, :] + acc_v[1, s, :])
                           + (acc_v[2, s, :] + acc_v[3, s, :]))
        pltpu.sync_copy(out_v, out_hbm.at[wid])

    return body(feat2d, intr2d, divs, segs, vecs, intr_w, intr_b)


def _sum_body(x_ref, o_ref):
    o_ref[...] = jnp.sum(jnp.sum(x_ref[...], axis=2), axis=0, keepdims=True)


def kernel(intr_idxs, intr_divs, feat_idxs, segment_ids, vecs, intr_W, intr_b):
    feat2d = feat_idxs.reshape(2 * T // GROUP, GROUP)
    intr2d = intr_idxs.reshape(T // GROUP, GROUP)
    partials = _sc_partials(feat2d, intr2d, intr_divs, segment_ids,
                            vecs, intr_W.reshape(-1), jnp.tile(intr_b, 16))
    out = pl.pallas_call(
        _sum_body,
        out_shape=jax.ShapeDtypeStruct((1, B), jnp.float32),
    )(partials)
    return out[0]
